# baseline (speedup 1.0000x reference)
import jax
import jax.numpy as jnp
from jax.experimental import pallas as pl


def _ew_kernel(x_ref, o_ref):
    o_ref[...] = x_ref[...] * 32.0 + 64.0


def kernel(input_xyzs):
    n, c = input_xyzs.shape
    total = n * c
    lanes = 128
    rows = total // lanes
    flat = input_xyzs.reshape(rows, lanes)
    out = pl.pallas_call(
        _ew_kernel,
        out_shape=jax.ShapeDtypeStruct((rows, lanes), input_xyzs.dtype),
    )(flat)
    return out.reshape(n, c)


# speedup vs baseline: 1.0292x; 1.0292x over previous
import jax
import jax.numpy as jnp
from jax.experimental import pallas as pl


def _ew_kernel(x_ref, o_ref):
    o_ref[...] = x_ref[...] * 32.0 + 64.0


def kernel(input_xyzs):
    return pl.pallas_call(
        _ew_kernel,
        out_shape=jax.ShapeDtypeStruct(input_xyzs.shape, input_xyzs.dtype),
    )(input_xyzs)


# speedup vs baseline: 5.9714x; 5.8018x over previous
import jax
import jax.numpy as jnp
from jax.experimental import pallas as pl


def _ew_kernel(x_ref, o_ref):
    o_ref[...] = x_ref[...] * 32.0 + 64.0


def kernel(input_xyzs):
    xt = input_xyzs.T
    out = pl.pallas_call(
        _ew_kernel,
        out_shape=jax.ShapeDtypeStruct(xt.shape, xt.dtype),
    )(xt)
    return out.T
